# baseline (speedup 1.0000x reference)
import functools

import jax
import jax.numpy as jnp
from jax import lax
from jax.experimental import pallas as pl
from jax.experimental.pallas import tpu as pltpu
from jax.experimental.pallas import tpu_sc as plsc

NC, NS = 2, 16
NW = NC * NS
CHUNK = 128
G = 4
B = 4096 * 200
PW = B // NW
NCHUNK = PW // CHUNK
SCH = NCHUNK // G
D = 64

_mesh = plsc.VectorSubcoreMesh(
    core_axis_name="c", subcore_axis_name="s", num_cores=NC, num_subcores=NS
)


@functools.partial(
    pl.kernel,
    out_type=(
        jax.ShapeDtypeStruct((NW, NCHUNK, CHUNK, D), jnp.float32),
        jax.ShapeDtypeStruct((NW, NCHUNK, CHUNK, D), jnp.float32),
    ),
    mesh=_mesh,
    compiler_params=pltpu.CompilerParams(use_tc_tiling_on_sc=False),
    scratch_types=[
        pltpu.VMEM((NCHUNK, CHUNK), jnp.int32),
        pltpu.VMEM((G, CHUNK, D), jnp.float32),
        pltpu.VMEM((G, CHUNK, D), jnp.float32),
        pltpu.SemaphoreType.DMA,
        pltpu.SemaphoreType.DMA,
        pltpu.SemaphoreType.DMA,
        pltpu.SemaphoreType.DMA,
    ],
)
def _emb_kernel(hour_idx, day_idx, hour_tab, day_tab, out_hour, out_day,
                idx_v, buf_a, buf_b, gsem_a, gsem_b, wsem_a, wsem_b):
    wid = lax.axis_index("s") * NC + lax.axis_index("c")

    def phase(idx_hbm, tab, out):
        pltpu.sync_copy(idx_hbm.at[wid], idx_v)

        def start_gather(s, buf, sem):
            for j in range(G):
                pltpu.async_copy(tab.at[idx_v.at[s * G + j]], buf.at[j], sem)

        def wait_gather(buf, sem):
            pltpu.make_async_copy(out.at[wid, pl.ds(0, G)], buf, sem).wait()

        def start_write(s, buf, sem):
            pltpu.async_copy(buf, out.at[wid, pl.ds(s * G, G)], sem)

        def wait_write(buf, sem):
            pltpu.make_async_copy(buf, out.at[wid, pl.ds(0, G)], sem).wait()

        start_gather(0, buf_a, gsem_a)

        def body(i, carry):
            s = 2 * i

            @pl.when(i > 0)
            def _():
                wait_write(buf_b, wsem_b)
            start_gather(s + 1, buf_b, gsem_b)
            wait_gather(buf_a, gsem_a)
            start_write(s, buf_a, wsem_a)
            wait_gather(buf_b, gsem_b)
            start_write(s + 1, buf_b, wsem_b)
            wait_write(buf_a, wsem_a)

            @pl.when(s + 2 < SCH)
            def _():
                start_gather(s + 2, buf_a, gsem_a)
            return carry

        lax.fori_loop(0, SCH // 2, body, 0)
        wait_write(buf_b, wsem_b)

    phase(hour_idx, hour_tab, out_hour)
    phase(day_idx, day_tab, out_day)


def kernel(t_hour, t_day, time_in_day_table, day_in_week_table):
    S, T = t_hour.shape
    h = t_hour.astype(jnp.int32).reshape(NW, NCHUNK, CHUNK)
    d = t_day.astype(jnp.int32).reshape(NW, NCHUNK, CHUNK)
    oh, od = _emb_kernel(h, d, time_in_day_table, day_in_week_table)
    return oh.reshape(S, T, D), od.reshape(S, T, D)


# speedup vs baseline: 1.3300x; 1.3300x over previous
import functools

import jax
import jax.numpy as jnp
from jax import lax
from jax.experimental import pallas as pl
from jax.experimental.pallas import tpu as pltpu
from jax.experimental.pallas import tpu_sc as plsc

NC, NS = 2, 16
NW = NC * NS
L = 16
B = 4096 * 200
PW = B // NW
CH = 256
NST = PW // CH
D = 64
VH = 288
VD = 7

_mesh = plsc.VectorSubcoreMesh(
    core_axis_name="c", subcore_axis_name="s", num_cores=NC, num_subcores=NS
)


@functools.partial(
    pl.kernel,
    out_type=(
        jax.ShapeDtypeStruct((NW, NST, CH * D), jnp.float32),
        jax.ShapeDtypeStruct((NW, NST, CH * D), jnp.float32),
    ),
    mesh=_mesh,
    compiler_params=pltpu.CompilerParams(
        use_tc_tiling_on_sc=False, needs_layout_passes=False
    ),
    scratch_types=[
        pltpu.VMEM((VH * D,), jnp.float32),
        pltpu.VMEM((VD * D,), jnp.float32),
        pltpu.VMEM((PW,), jnp.int32),
        pltpu.VMEM((CH * D,), jnp.float32),
        pltpu.VMEM((CH * D,), jnp.float32),
        pltpu.SemaphoreType.DMA,
        pltpu.SemaphoreType.DMA,
    ],
)
def _emb_kernel(hour_idx, day_idx, hour_tab, day_tab, out_hour, out_day,
                tab_h, tab_d, idx_v, stage_a, stage_b, wsem_a, wsem_b):
    wid = lax.axis_index("s") * NC + lax.axis_index("c")
    pltpu.sync_copy(hour_tab, tab_h)
    pltpu.sync_copy(day_tab, tab_d)
    iota64 = lax.iota(jnp.int32, L) * D

    def fill(stage, tab, s):
        def kbody(k, carry):
            idx16 = idx_v[pl.ds(s * CH + k * L, L)]
            base = idx16 * D
            obase = jnp.full((L,), k * (L * D), jnp.int32) + iota64
            for c in range(D):
                v = plsc.load_gather(tab, [base + c])
                plsc.store_scatter(stage, [obase + c], v)
            return carry

        lax.fori_loop(0, CH // L, kbody, 0)

    def wait_write(stage, out, sem):
        pltpu.make_async_copy(stage, out.at[wid, 0], sem).wait()

    def phase(idx_hbm, tab, out):
        pltpu.sync_copy(idx_hbm.at[wid], idx_v)

        def body(i, carry):
            s = 2 * i

            @pl.when(i > 0)
            def _():
                wait_write(stage_a, out, wsem_a)
            fill(stage_a, tab, s)
            pltpu.async_copy(stage_a, out.at[wid, s], wsem_a)

            @pl.when(i > 0)
            def _():
                wait_write(stage_b, out, wsem_b)
            fill(stage_b, tab, s + 1)
            pltpu.async_copy(stage_b, out.at[wid, s + 1], wsem_b)
            return carry

        lax.fori_loop(0, NST // 2, body, 0)
        wait_write(stage_a, out, wsem_a)
        wait_write(stage_b, out, wsem_b)

    phase(hour_idx, tab_h, out_hour)
    phase(day_idx, tab_d, out_day)


def kernel(t_hour, t_day, time_in_day_table, day_in_week_table):
    S, T = t_hour.shape
    h = t_hour.astype(jnp.int32).reshape(NW, PW)
    d = t_day.astype(jnp.int32).reshape(NW, PW)
    oh, od = _emb_kernel(h, d, time_in_day_table.reshape(VH * D),
                         day_in_week_table.reshape(VD * D))
    return oh.reshape(S, T, D), od.reshape(S, T, D)


# speedup vs baseline: 1.7289x; 1.2999x over previous
import functools

import jax
import jax.numpy as jnp
from jax import lax
from jax.experimental import pallas as pl
from jax.experimental.pallas import tpu as pltpu
from jax.experimental.pallas import tpu_sc as plsc

NC, NS = 2, 16
NW = NC * NS
L = 16
B = 4096 * 200
PW = B // NW
CH = 256
NST = PW // CH
D = 64
VH = 288
VD = 7

_mesh = plsc.VectorSubcoreMesh(
    core_axis_name="c", subcore_axis_name="s", num_cores=NC, num_subcores=NS
)


@functools.partial(
    pl.kernel,
    out_type=(
        jax.ShapeDtypeStruct((NW, NST, CH * D), jnp.float32),
        jax.ShapeDtypeStruct((NW, NST, CH * D), jnp.float32),
    ),
    mesh=_mesh,
    compiler_params=pltpu.CompilerParams(
        use_tc_tiling_on_sc=False, needs_layout_passes=False
    ),
    scratch_types=[
        pltpu.VMEM((VH * D,), jnp.float32),
        pltpu.VMEM((VD * D,), jnp.float32),
        pltpu.VMEM((PW,), jnp.int32),
        pltpu.VMEM((CH * D,), jnp.float32),
        pltpu.VMEM((CH * D,), jnp.float32),
        pltpu.SemaphoreType.DMA,
        pltpu.SemaphoreType.DMA,
    ],
)
def _emb_kernel(hour_idx, day_idx, hour_tab, day_tab, out_hour, out_day,
                tab_h, tab_d, idx_v, stage_a, stage_b, wsem_a, wsem_b):
    wid = lax.axis_index("s") * NC + lax.axis_index("c")
    pltpu.sync_copy(hour_tab, tab_h)
    pltpu.sync_copy(day_tab, tab_d)
    iota64 = lax.iota(jnp.int32, L) * D

    def fill(stage, tab, s):
        def kbody(k, carry):
            idx16 = idx_v[pl.ds(s * CH + k * L, L)]
            base = idx16 * D
            obase = jnp.full((L,), k * (L * D), jnp.int32) + iota64
            for g in range(0, D, L):
                vs = [plsc.load_gather(tab, [base + (g + j)]) for j in range(L)]
                for j in range(L):
                    plsc.store_scatter(stage, [obase + (g + j)], vs[j])
            return carry

        lax.fori_loop(0, CH // L, kbody, 0)

    def wait_write(stage, out, sem):
        pltpu.make_async_copy(stage, out.at[wid, 0], sem).wait()

    def phase(idx_hbm, tab, out):
        pltpu.sync_copy(idx_hbm.at[wid], idx_v)

        def body(i, carry):
            s = 2 * i

            @pl.when(i > 0)
            def _():
                wait_write(stage_a, out, wsem_a)
            fill(stage_a, tab, s)
            pltpu.async_copy(stage_a, out.at[wid, s], wsem_a)

            @pl.when(i > 0)
            def _():
                wait_write(stage_b, out, wsem_b)
            fill(stage_b, tab, s + 1)
            pltpu.async_copy(stage_b, out.at[wid, s + 1], wsem_b)
            return carry

        lax.fori_loop(0, NST // 2, body, 0)
        wait_write(stage_a, out, wsem_a)
        wait_write(stage_b, out, wsem_b)

    phase(hour_idx, tab_h, out_hour)
    phase(day_idx, tab_d, out_day)


def kernel(t_hour, t_day, time_in_day_table, day_in_week_table):
    S, T = t_hour.shape
    h = t_hour.astype(jnp.int32).reshape(NW, PW)
    d = t_day.astype(jnp.int32).reshape(NW, PW)
    oh, od = _emb_kernel(h, d, time_in_day_table.reshape(VH * D),
                         day_in_week_table.reshape(VD * D))
    return oh.reshape(S, T, D), od.reshape(S, T, D)


# speedup vs baseline: 2.0147x; 1.1653x over previous
import functools

import jax
import jax.numpy as jnp
from jax import lax
from jax.experimental import pallas as pl
from jax.experimental.pallas import tpu as pltpu
from jax.experimental.pallas import tpu_sc as plsc

NC, NS = 2, 16
NW = NC * NS
L = 16
B = 4096 * 200
PW = B // NW
CH = 256
NST = PW // CH
D = 64
VH = 288
VD = 7

_mesh = plsc.VectorSubcoreMesh(
    core_axis_name="c", subcore_axis_name="s", num_cores=NC, num_subcores=NS
)


@functools.partial(
    pl.kernel,
    out_type=(
        jax.ShapeDtypeStruct((NW, NST, CH * D), jnp.float32),
        jax.ShapeDtypeStruct((NW, NST, CH * D), jnp.float32),
    ),
    mesh=_mesh,
    compiler_params=pltpu.CompilerParams(
        use_tc_tiling_on_sc=False, needs_layout_passes=False
    ),
    scratch_types=[
        pltpu.VMEM((VH * D,), jnp.float32),
        pltpu.VMEM((VD * D,), jnp.float32),
        pltpu.VMEM((PW,), jnp.int32),
        pltpu.VMEM((CH * D,), jnp.float32),
        pltpu.VMEM((CH * D,), jnp.float32),
        pltpu.SemaphoreType.DMA,
        pltpu.SemaphoreType.DMA,
    ],
)
def _emb_kernel(hour_idx, day_idx, hour_tab, day_tab, out_hour, out_day,
                tab_h, tab_d, idx_v, stage_a, stage_b, wsem_a, wsem_b):
    wid = lax.axis_index("s") * NC + lax.axis_index("c")
    pltpu.sync_copy(hour_tab, tab_h)
    pltpu.sync_copy(day_tab, tab_d)
    iota64 = lax.iota(jnp.int32, L) * D

    def fill(stage, tab, s):
        @plsc.parallel_loop(0, CH // L)
        def kbody(k):
            idx16 = idx_v[pl.ds(s * CH + k * L, L)]
            base = idx16 * D
            obase = jnp.full((L,), k * (L * D), jnp.int32) + iota64
            for g in range(0, D, L):
                vs = [plsc.load_gather(tab, [base + (g + j)]) for j in range(L)]
                for j in range(L):
                    plsc.store_scatter(stage, [obase + (g + j)], vs[j])

    def wait_write(stage, out, sem):
        pltpu.make_async_copy(stage, out.at[wid, 0], sem).wait()

    def phase(idx_hbm, tab, out):
        pltpu.sync_copy(idx_hbm.at[wid], idx_v)

        def body(i, carry):
            s = 2 * i

            @pl.when(i > 0)
            def _():
                wait_write(stage_a, out, wsem_a)
            fill(stage_a, tab, s)
            pltpu.async_copy(stage_a, out.at[wid, s], wsem_a)

            @pl.when(i > 0)
            def _():
                wait_write(stage_b, out, wsem_b)
            fill(stage_b, tab, s + 1)
            pltpu.async_copy(stage_b, out.at[wid, s + 1], wsem_b)
            return carry

        lax.fori_loop(0, NST // 2, body, 0)
        wait_write(stage_a, out, wsem_a)
        wait_write(stage_b, out, wsem_b)

    phase(hour_idx, tab_h, out_hour)
    phase(day_idx, tab_d, out_day)


def kernel(t_hour, t_day, time_in_day_table, day_in_week_table):
    S, T = t_hour.shape
    h = t_hour.astype(jnp.int32).reshape(NW, PW)
    d = t_day.astype(jnp.int32).reshape(NW, PW)
    oh, od = _emb_kernel(h, d, time_in_day_table.reshape(VH * D),
                         day_in_week_table.reshape(VD * D))
    return oh.reshape(S, T, D), od.reshape(S, T, D)


# speedup vs baseline: 5.3714x; 2.6661x over previous
import functools

import jax
import jax.numpy as jnp
from jax import lax
from jax.experimental import pallas as pl
from jax.experimental.pallas import tpu as pltpu
from jax.experimental.pallas import tpu_sc as plsc

NC, NS = 2, 16
NW = NC * NS
L = 16
B = 4096 * 200
PW = B // NW
CH = 256
NST = PW // CH
D = 64
VH = 288
VD = 7

_mesh = plsc.VectorSubcoreMesh(
    core_axis_name="c", subcore_axis_name="s", num_cores=NC, num_subcores=NS
)


@functools.partial(
    pl.kernel,
    out_type=(
        jax.ShapeDtypeStruct((NW, NST, CH * D), jnp.float32),
        jax.ShapeDtypeStruct((NW, NST, CH * D), jnp.float32),
    ),
    mesh=_mesh,
    compiler_params=pltpu.CompilerParams(
        use_tc_tiling_on_sc=False, needs_layout_passes=False
    ),
    scratch_types=[
        pltpu.VMEM((VH * D,), jnp.float32),
        pltpu.VMEM((VD * D,), jnp.float32),
        pltpu.VMEM((PW,), jnp.int32),
        pltpu.VMEM((CH * D,), jnp.float32),
        pltpu.VMEM((CH * D,), jnp.float32),
        pltpu.SemaphoreType.DMA,
        pltpu.SemaphoreType.DMA,
    ],
)
def _emb_kernel(hour_idx, day_idx, hour_tab, day_tab, out_hour, out_day,
                tab_h, tab_d, idx_v, stage_a, stage_b, wsem_a, wsem_b):
    wid = lax.axis_index("s") * NC + lax.axis_index("c")
    pltpu.sync_copy(hour_tab, tab_h)
    pltpu.sync_copy(day_tab, tab_d)
    iota64 = lax.iota(jnp.int32, L) * D

    def fill(stage, tab, s):
        @plsc.parallel_loop(0, CH // L)
        def kbody(k):
            idx16 = idx_v[pl.ds(s * CH + k * L, L)] * D
            for j in range(L):
                b = idx16[j]
                for g in range(0, D, L):
                    stage[pl.ds((k * L + j) * D + g, L)] = tab[pl.ds(b + g, L)]

    def wait_write(stage, out, sem):
        pltpu.make_async_copy(stage, out.at[wid, 0], sem).wait()

    def phase(idx_hbm, tab, out):
        pltpu.sync_copy(idx_hbm.at[wid], idx_v)

        def body(i, carry):
            s = 2 * i

            @pl.when(i > 0)
            def _():
                wait_write(stage_a, out, wsem_a)
            fill(stage_a, tab, s)
            pltpu.async_copy(stage_a, out.at[wid, s], wsem_a)

            @pl.when(i > 0)
            def _():
                wait_write(stage_b, out, wsem_b)
            fill(stage_b, tab, s + 1)
            pltpu.async_copy(stage_b, out.at[wid, s + 1], wsem_b)
            return carry

        lax.fori_loop(0, NST // 2, body, 0)
        wait_write(stage_a, out, wsem_a)
        wait_write(stage_b, out, wsem_b)

    phase(hour_idx, tab_h, out_hour)
    phase(day_idx, tab_d, out_day)


def kernel(t_hour, t_day, time_in_day_table, day_in_week_table):
    S, T = t_hour.shape
    h = t_hour.astype(jnp.int32).reshape(NW, PW)
    d = t_day.astype(jnp.int32).reshape(NW, PW)
    oh, od = _emb_kernel(h, d, time_in_day_table.reshape(VH * D),
                         day_in_week_table.reshape(VD * D))
    return oh.reshape(S, T, D), od.reshape(S, T, D)


# speedup vs baseline: 5.4128x; 1.0077x over previous
import functools

import jax
import jax.numpy as jnp
from jax import lax
from jax.experimental import pallas as pl
from jax.experimental.pallas import tpu as pltpu
from jax.experimental.pallas import tpu_sc as plsc

NC, NS = 2, 16
NW = NC * NS
L = 16
B = 4096 * 200
PW = B // NW
CH = 256
NST = PW // CH
D = 64
IDXR = PW // 128

_mesh = plsc.VectorSubcoreMesh(
    core_axis_name="c", subcore_axis_name="s", num_cores=NC, num_subcores=NS
)


@functools.partial(
    pl.kernel,
    out_type=(
        jax.ShapeDtypeStruct((NW * NST * 128, 128), jnp.float32),
        jax.ShapeDtypeStruct((NW * NST * 128, 128), jnp.float32),
    ),
    mesh=_mesh,
    scratch_types=[
        pltpu.VMEM((144, 128), jnp.float32),
        pltpu.VMEM((8, 128), jnp.float32),
        pltpu.VMEM((IDXR, 128), jnp.int32),
        pltpu.VMEM((128, 128), jnp.float32),
        pltpu.VMEM((128, 128), jnp.float32),
        pltpu.SemaphoreType.DMA,
        pltpu.SemaphoreType.DMA,
    ],
)
def _emb_kernel(hour_idx, day_idx, hour_tab, day_tab, out_hour, out_day,
                tab_h, tab_d, idx_v, stage_a, stage_b, wsem_a, wsem_b):
    wid = lax.axis_index("s") * NC + lax.axis_index("c")
    pltpu.sync_copy(hour_tab, tab_h)
    pltpu.sync_copy(day_tab, tab_d)

    def fill(stage, tab, s):
        @plsc.parallel_loop(0, CH // L)
        def kbody(k):
            idx16 = idx_v[2 * s + k // 8, pl.ds((k % 8) * L, L)]
            for j in range(L):
                e = idx16[j]
                trow = e >> 1
                tcol = (e & 1) << 6
                for g in range(0, D, L):
                    stage[k * 8 + j // 2, pl.ds((j % 2) * D + g, L)] = (
                        tab[trow, pl.ds(tcol + g, L)]
                    )

    def out_block(out, s):
        return out.at[pl.ds((wid * NST + s) * 128, 128)]

    def wait_write(stage, out, sem):
        pltpu.make_async_copy(stage, out_block(out, 0), sem).wait()

    def phase(idx_hbm, tab, out):
        pltpu.sync_copy(idx_hbm.at[pl.ds(wid * IDXR, IDXR)], idx_v)

        def body(i, carry):
            s = 2 * i

            @pl.when(i > 0)
            def _():
                wait_write(stage_a, out, wsem_a)
            fill(stage_a, tab, s)
            pltpu.async_copy(stage_a, out_block(out, s), wsem_a)

            @pl.when(i > 0)
            def _():
                wait_write(stage_b, out, wsem_b)
            fill(stage_b, tab, s + 1)
            pltpu.async_copy(stage_b, out_block(out, s + 1), wsem_b)
            return carry

        lax.fori_loop(0, NST // 2, body, 0)
        wait_write(stage_a, out, wsem_a)
        wait_write(stage_b, out, wsem_b)

    phase(hour_idx, tab_h, out_hour)
    phase(day_idx, tab_d, out_day)


def kernel(t_hour, t_day, time_in_day_table, day_in_week_table):
    S, T = t_hour.shape
    h = t_hour.astype(jnp.int32).reshape(NW * IDXR, 128)
    d = t_day.astype(jnp.int32).reshape(NW * IDXR, 128)
    th = time_in_day_table.reshape(144, 128)
    td = jnp.pad(day_in_week_table, ((0, 9), (0, 0))).reshape(8, 128)
    oh, od = _emb_kernel(h, d, th, td)
    return oh.reshape(S, T, D), od.reshape(S, T, D)


# speedup vs baseline: 7.4393x; 1.3744x over previous
import functools

import jax
import jax.numpy as jnp
from jax import lax
from jax.experimental import pallas as pl
from jax.experimental.pallas import tpu as pltpu
from jax.experimental.pallas import tpu_sc as plsc

NC, NS = 2, 16
NW = NC * NS
L = 16
S0, T0 = 4096, 200
PW = (S0 // NW) * T0
BPW = S0 // NW
D = 64

_mesh = plsc.VectorSubcoreMesh(
    core_axis_name="c", subcore_axis_name="s", num_cores=NC, num_subcores=NS
)


def _make_lookup(tab_rows):
    @functools.partial(
        pl.kernel,
        out_type=jax.ShapeDtypeStruct((S0, T0, D), jnp.float32),
        mesh=_mesh,
        scratch_types=[
            pltpu.VMEM((tab_rows, 128), jnp.float32),
            pltpu.VMEM((PW,), jnp.int32),
            pltpu.VMEM((T0, D), jnp.float32),
            pltpu.VMEM((T0, D), jnp.float32),
            pltpu.SemaphoreType.DMA,
            pltpu.SemaphoreType.DMA,
        ],
    )
    def _lookup(idx_hbm, tab_hbm, out, tab_v, idx_v, stage_a, stage_b,
                wsem_a, wsem_b):
        wid = lax.axis_index("s") * NC + lax.axis_index("c")
        pltpu.sync_copy(tab_hbm, tab_v)
        pltpu.sync_copy(idx_hbm.at[pl.ds(wid * PW, PW)], idx_v)
        b0 = wid * BPW

        def copy_row(stage, e, t):
            trow = e >> 1
            tcol = (e & 1) << 6
            for g in range(0, D, L):
                stage[t, pl.ds(g, L)] = tab_v[trow, pl.ds(tcol + g, L)]

        def fill(stage, i):
            p0 = i * T0

            @plsc.parallel_loop(0, T0 // L)
            def kbody(k):
                idx16 = idx_v[pl.ds(p0 + k * L, L)]
                for j in range(L):
                    copy_row(stage, idx16[j], k * L + j)

            idx16 = idx_v[pl.ds(p0 + T0 - L, L)]
            for j in range(L // 2, L):
                copy_row(stage, idx16[j], T0 - L + j)

        def wait_write(stage, sem):
            pltpu.make_async_copy(stage, out.at[0], sem).wait()

        def body(i, carry):
            @pl.when(i > 0)
            def _():
                wait_write(stage_a, wsem_a)
            fill(stage_a, 2 * i)
            pltpu.async_copy(stage_a, out.at[b0 + 2 * i], wsem_a)

            @pl.when(i > 0)
            def _():
                wait_write(stage_b, wsem_b)
            fill(stage_b, 2 * i + 1)
            pltpu.async_copy(stage_b, out.at[b0 + 2 * i + 1], wsem_b)
            return carry

        lax.fori_loop(0, BPW // 2, body, 0)
        wait_write(stage_a, wsem_a)
        wait_write(stage_b, wsem_b)

    return _lookup


_lookup_hour = _make_lookup(144)
_lookup_day = _make_lookup(8)


def kernel(t_hour, t_day, time_in_day_table, day_in_week_table):
    h = t_hour.astype(jnp.int32).reshape(S0 * T0)
    d = t_day.astype(jnp.int32).reshape(S0 * T0)
    th = time_in_day_table.reshape(144, 128)
    td = jnp.pad(day_in_week_table, ((0, 9), (0, 0))).reshape(8, 128)
    return _lookup_hour(h, th), _lookup_day(d, td)
